# Initial kernel scaffold; baseline (speedup 1.0000x reference)
import jax
import jax.numpy as jnp
from jax.experimental import pallas as pl


def kernel(pe, t):
    raise NotImplementedError("write your pallas kernel here")


# speedup vs baseline: 1.9492x; 1.9492x over previous
import functools

import jax
import jax.numpy as jnp
from jax import lax
from jax.experimental import pallas as pl
from jax.experimental.pallas import tpu as pltpu
from jax.experimental.pallas import tpu_sc as plsc

D_MODEL = 1024
TIME_STEPS = 8192
BATCH = 16384

_info = plsc.get_sparse_core_info()
_NC = _info.num_cores
_NS = _info.num_subcores
_NW = _NC * _NS
_BPW = BATCH // _NW
_CHUNK = 64
_NCHUNK = _BPW // _CHUNK

_mesh = plsc.VectorSubcoreMesh(core_axis_name="c", subcore_axis_name="s")


@functools.partial(
    pl.kernel,
    mesh=_mesh,
    out_type=jax.ShapeDtypeStruct((BATCH, D_MODEL), jnp.float32),
    scratch_types=[
        pltpu.VMEM((_BPW,), jnp.int32),
        pltpu.VMEM((_CHUNK, D_MODEL), jnp.float32),
        pltpu.SemaphoreType.DMA,
    ],
)
def _gather_kernel(pe_hbm, t_hbm, out_hbm, idx_v, rows_v, sem):
    wid = lax.axis_index("s") * _NC + lax.axis_index("c")
    base = wid * _BPW
    pltpu.sync_copy(t_hbm.at[pl.ds(base, _BPW)], idx_v)
    for c in range(_NCHUNK):
        idx_slice = idx_v.at[pl.ds(c * _CHUNK, _CHUNK)]
        pltpu.async_copy(pe_hbm.at[idx_slice], rows_v, sem).wait()
        pltpu.sync_copy(rows_v, out_hbm.at[pl.ds(base + c * _CHUNK, _CHUNK)])


def kernel(pe, t):
    return _gather_kernel(pe, t)


# speedup vs baseline: 2.0447x; 1.0490x over previous
import functools

import jax
import jax.numpy as jnp
from jax import lax
from jax.experimental import pallas as pl
from jax.experimental.pallas import tpu as pltpu
from jax.experimental.pallas import tpu_sc as plsc

D_MODEL = 1024
TIME_STEPS = 8192
BATCH = 16384

_info = plsc.get_sparse_core_info()
_NC = _info.num_cores
_NS = _info.num_subcores
_NW = _NC * _NS
_BPW = BATCH // _NW
_CHUNK = 32
_NCHUNK = _BPW // _CHUNK

_mesh = plsc.VectorSubcoreMesh(core_axis_name="c", subcore_axis_name="s")


@functools.partial(
    pl.kernel,
    mesh=_mesh,
    out_type=jax.ShapeDtypeStruct((BATCH, D_MODEL), jnp.float32),
    scratch_types=[
        pltpu.VMEM((_BPW,), jnp.int32),
        pltpu.VMEM((_CHUNK, D_MODEL), jnp.float32),
        pltpu.VMEM((_CHUNK, D_MODEL), jnp.float32),
        pltpu.SemaphoreType.DMA,
        pltpu.SemaphoreType.DMA,
        pltpu.SemaphoreType.DMA,
        pltpu.SemaphoreType.DMA,
    ],
)
def _gather_kernel(pe_hbm, t_hbm, out_hbm, idx_v, rows0, rows1, g0, g1, w0, w1):
    wid = lax.axis_index("s") * _NC + lax.axis_index("c")
    base = wid * _BPW
    rows = (rows0, rows1)
    gsem = (g0, g1)
    wsem = (w0, w1)
    pltpu.sync_copy(t_hbm.at[pl.ds(base, _BPW)], idx_v)

    def gather(c):
        idx_slice = idx_v.at[pl.ds(c * _CHUNK, _CHUNK)]
        return pltpu.async_copy(pe_hbm.at[idx_slice], rows[c % 2], gsem[c % 2])

    def writeback(c):
        dst = out_hbm.at[pl.ds(base + c * _CHUNK, _CHUNK)]
        return pltpu.async_copy(rows[c % 2], dst, wsem[c % 2])

    g = [None, None]
    w = [None, None]
    g[0] = gather(0)
    for c in range(_NCHUNK):
        nxt = (c + 1) % 2
        if w[nxt] is not None:
            w[nxt].wait()
            w[nxt] = None
        if c + 1 < _NCHUNK:
            g[nxt] = gather(c + 1)
        g[c % 2].wait()
        w[c % 2] = writeback(c)
    w[(_NCHUNK - 1) % 2].wait()


def kernel(pe, t):
    return _gather_kernel(pe, t)


# speedup vs baseline: 2.0718x; 1.0133x over previous
import functools

import jax
import jax.numpy as jnp
from jax import lax
from jax.experimental import pallas as pl
from jax.experimental.pallas import tpu as pltpu
from jax.experimental.pallas import tpu_sc as plsc

D_MODEL = 1024
TIME_STEPS = 8192
BATCH = 16384

_info = plsc.get_sparse_core_info()
_NC = _info.num_cores
_NS = _info.num_subcores
_NW = _NC * _NS
_BPW = BATCH // _NW
_CHUNK = 32
_NCHUNK = _BPW // _CHUNK
_NBUF = 3

_mesh = plsc.VectorSubcoreMesh(core_axis_name="c", subcore_axis_name="s")


@functools.partial(
    pl.kernel,
    mesh=_mesh,
    out_type=jax.ShapeDtypeStruct((BATCH, D_MODEL), jnp.float32),
    scratch_types=[
        pltpu.VMEM((_BPW,), jnp.int32),
    ]
    + [pltpu.VMEM((_CHUNK, D_MODEL), jnp.float32) for _ in range(_NBUF)]
    + [pltpu.SemaphoreType.DMA for _ in range(2 * _NBUF)],
)
def _gather_kernel(pe_hbm, t_hbm, out_hbm, idx_v, *bufs):
    rows = bufs[:_NBUF]
    gsem = bufs[_NBUF : 2 * _NBUF]
    wsem = bufs[2 * _NBUF :]
    wid = lax.axis_index("s") * _NC + lax.axis_index("c")
    base = wid * _BPW
    pltpu.sync_copy(t_hbm.at[pl.ds(base, _BPW)], idx_v)

    def gather(c):
        b = c % _NBUF
        idx_slice = idx_v.at[pl.ds(c * _CHUNK, _CHUNK)]
        return pltpu.async_copy(pe_hbm.at[idx_slice], rows[b], gsem[b])

    def writeback(c):
        b = c % _NBUF
        dst = out_hbm.at[pl.ds(base + c * _CHUNK, _CHUNK)]
        return pltpu.async_copy(rows[b], dst, wsem[b])

    g = [None] * _NBUF
    w = [None] * _NBUF
    for c in range(_NBUF - 1):
        g[c % _NBUF] = gather(c)
    for c in range(_NCHUNK):
        b = c % _NBUF
        nxt = c + _NBUF - 1
        if nxt < _NCHUNK:
            nb = nxt % _NBUF
            if w[nb] is not None:
                w[nb].wait()
                w[nb] = None
            g[nb] = gather(nxt)
        g[b].wait()
        w[b] = writeback(c)
    for b in range(_NBUF):
        if w[b] is not None:
            w[b].wait()


def kernel(pe, t):
    return _gather_kernel(pe, t)
